# Initial kernel scaffold; baseline (speedup 1.0000x reference)
import jax
import jax.numpy as jnp
from jax.experimental import pallas as pl


def kernel(x, edge_index, W_gcn, b_gcn, W1, b1, W2, b2, W3, b3, Wo, bo):
    raise NotImplementedError("write your pallas kernel here")


# speedup vs baseline: 8.1136x; 8.1136x over previous
import functools

import jax
import jax.numpy as jnp
from jax import lax
from jax.experimental import pallas as pl
from jax.experimental.pallas import tpu as pltpu
from jax.experimental.pallas import tpu_sc as plsc

ALPHA = 0.1

_NC = 2
_NS = 16
_L = 16

_SL = 128
_EPT = 20480


def _make_deg_kernel(N, E):
    EPW = E // (_NC * _NS)
    mesh = plsc.VectorSubcoreMesh(core_axis_name="c", subcore_axis_name="s")

    @functools.partial(
        pl.kernel,
        out_type=jax.ShapeDtypeStruct((_NC, N), jnp.float32),
        mesh=mesh,
        compiler_params=pltpu.CompilerParams(needs_layout_passes=False),
        scratch_types=[
            pltpu.VMEM((EPW,), jnp.int32),
            pltpu.VMEM((EPW,), jnp.float32),
            pltpu.VMEM_SHARED((N,), jnp.float32),
        ],
    )
    def deg_k(col_hbm, ones_hbm, zeros_hbm, out_hbm, col_v, ones_v, deg_sh):
        c = lax.axis_index("c")
        s = lax.axis_index("s")
        base = (c * _NS + s) * EPW
        pltpu.sync_copy(col_hbm.at[pl.ds(base, EPW)], col_v)
        pltpu.sync_copy(ones_hbm, ones_v)

        @pl.when(s == 0)
        def _():
            pltpu.sync_copy(zeros_hbm, deg_sh)

        plsc.subcore_barrier()
        pltpu.sync_copy(ones_v, deg_sh.at[col_v], add=True)
        plsc.subcore_barrier()

        @pl.when(s == 0)
        def _():
            pltpu.sync_copy(deg_sh, out_hbm.at[c])

    return deg_k


def _make_scatter_kernel(N, E, D):
    NSL = D // _SL
    SW = NSL // _NC
    B = 128
    STG = 2048
    NSTG = _EPT // STG
    NB = STG // B
    AR = N + 8
    RS = 640
    LAST = N - (_NS - 1) * RS
    mesh = plsc.VectorSubcoreMesh(core_axis_name="c", subcore_axis_name="s")

    @functools.partial(
        pl.kernel,
        out_type=jax.ShapeDtypeStruct((NSL * N, _SL), jnp.float32),
        mesh=mesh,
        compiler_params=pltpu.CompilerParams(needs_layout_passes=False),
        scratch_types=[
            pltpu.VMEM((STG,), jnp.int32),
            pltpu.VMEM((STG,), jnp.int32),
            pltpu.VMEM((B, _SL), jnp.float32),
            pltpu.VMEM((B, _SL), jnp.float32),
            pltpu.VMEM_SHARED((AR, _SL), jnp.float32),
            pltpu.SemaphoreType.DMA,
            pltpu.SemaphoreType.DMA,
        ],
    )
    def scat_k(row_hbm, col_hbm, hp_hbm, out_hbm,
               er_v, ec_v, rows_a, rows_b, acc_sh, sem_a, sem_b):
        c = lax.axis_index("c")
        s = lax.axis_index("s")
        zero16f = jnp.zeros((_L,), jnp.float32)

        for w in range(SW):
            sl_id = w * _NC + c

            def zb(i, _):
                for k in range(_SL // _L):
                    rows_a[i, pl.ds(k * _L, _L)] = zero16f
                return 0
            lax.fori_loop(0, B, zb, 0)

            @pl.when(s < _NS - 1)
            def _():
                for t in range(RS // B):
                    pltpu.sync_copy(rows_a, acc_sh.at[pl.ds(s * RS + t * B, B)])

            @pl.when(s == _NS - 1)
            def _():
                base = (_NS - 1) * RS
                nlast = LAST + 8
                for t in range(nlast // B):
                    pltpu.sync_copy(rows_a, acc_sh.at[pl.ds(base + t * B, B)])
                if nlast % B:
                    pltpu.sync_copy(
                        rows_a.at[pl.ds(0, nlast % B)],
                        acc_sh.at[pl.ds(base + (nlast // B) * B, nlast % B)])

            plsc.subcore_barrier()

            off16 = jnp.zeros((_L,), jnp.int32) + sl_id * N

            def sb(st, _):
                blk = s * _EPT + st * STG
                pltpu.sync_copy(row_hbm.at[pl.ds(blk, STG)], er_v)
                pltpu.sync_copy(col_hbm.at[pl.ds(blk, STG)], ec_v)

                def ob(i, _):
                    er_v[pl.ds(i * _L, _L)] = er_v[pl.ds(i * _L, _L)] + off16
                    return 0
                lax.fori_loop(0, STG // _L, ob, 0)

                bufs = (rows_a, rows_b)
                sems = (sem_a, sem_b)
                d = [None, None]
                d[0] = pltpu.async_copy(
                    hp_hbm.at[er_v.at[pl.ds(0, B)]], rows_a, sem_a)
                for k in range(NB):
                    p = k % 2
                    d[p].wait()
                    if k + 1 < NB:
                        q = (k + 1) % 2
                        d[q] = pltpu.async_copy(
                            hp_hbm.at[er_v.at[pl.ds((k + 1) * B, B)]],
                            bufs[q], sems[q])
                    pltpu.sync_copy(bufs[p],
                                    acc_sh.at[ec_v.at[pl.ds(k * B, B)]],
                                    add=True)
                return 0

            lax.fori_loop(0, NSTG, sb, 0)
            plsc.subcore_barrier()

            obase = sl_id * N

            @pl.when(s < _NS - 1)
            def _():
                pltpu.sync_copy(acc_sh.at[pl.ds(s * RS, RS)],
                                out_hbm.at[pl.ds(obase + s * RS, RS)])

            @pl.when(s == _NS - 1)
            def _():
                pltpu.sync_copy(acc_sh.at[pl.ds((_NS - 1) * RS, LAST)],
                                out_hbm.at[pl.ds(obase + (_NS - 1) * RS, LAST)])

            plsc.subcore_barrier()

    return scat_k


def _make_linear_kernel(N, D_IN, D):
    R = 1000
    NSL = D // _SL
    NR = N // R
    grid = (N // R, NSL)

    def body(x_ref, w_ref, degp_ref, hp_ref, dis_ref):
        deg = degp_ref[...]
        d = deg[:, 0:1] + deg[:, 1:2] + 1.0
        dis = lax.rsqrt(d)
        hp_ref[...] = jnp.dot(x_ref[...], w_ref[...],
                              preferred_element_type=jnp.float32) * dis
        dis_ref[...] = dis

    return pl.pallas_call(
        body,
        grid=grid,
        in_specs=[
            pl.BlockSpec((R, D_IN), lambda i, j: (i, 0)),
            pl.BlockSpec((D_IN, _SL), lambda i, j: (0, j)),
            pl.BlockSpec((R, 2), lambda i, j: (i, 0)),
        ],
        out_specs=[
            pl.BlockSpec((R, _SL), lambda i, j: (j * NR + i, 0)),
            pl.BlockSpec((R, 1), lambda i, j: (i, 0)),
        ],
        out_shape=[
            jax.ShapeDtypeStruct((NSL * N, _SL), jnp.float32),
            jax.ShapeDtypeStruct((N, 1), jnp.float32),
        ],
    )


def _make_mlp_kernel(N, D, DH):
    R = 1000
    NSL = D // _SL
    NR = N // R
    grid = (N // R,)

    def _leaky(z):
        return jnp.where(z >= 0, z, ALPHA * z)

    def body(a0, a1, a2, a3, h0, h1, h2, h3, dis_ref, bg_ref,
             w1_ref, b1_ref, w2_ref, b2_ref, w3_ref, b3_ref,
             wo_ref, bo_ref, out_ref):
        dis = dis_ref[...]
        acc = jnp.concatenate([a0[...], a1[...], a2[...], a3[...]], axis=1)
        hp = jnp.concatenate([h0[...], h1[...], h2[...], h3[...]], axis=1)
        conv = dis * (acc + hp) + bg_ref[...]
        z1 = _leaky(jnp.dot(conv, w1_ref[...],
                            preferred_element_type=jnp.float32) + b1_ref[...])
        z2 = _leaky(jnp.dot(z1, w2_ref[...],
                            preferred_element_type=jnp.float32) + b2_ref[...])
        z3 = _leaky(jnp.dot(z2, w3_ref[...],
                            preferred_element_type=jnp.float32) + b3_ref[...])
        out_ref[...] = jnp.dot(z3, wo_ref[...],
                               preferred_element_type=jnp.float32) + bo_ref[...]

    full = lambda a, b: pl.BlockSpec((a, b), lambda i: (0, 0))

    def slice_spec(k):
        return pl.BlockSpec((R, _SL), lambda i, k=k: (k * NR + i, 0))

    return pl.pallas_call(
        body,
        grid=grid,
        in_specs=[slice_spec(k) for k in range(NSL)]
        + [slice_spec(k) for k in range(NSL)]
        + [
            pl.BlockSpec((R, 1), lambda i: (i, 0)),
            full(1, D),
            full(D, DH), full(1, DH),
            full(DH, DH), full(1, DH),
            full(DH, DH), full(1, DH),
            full(DH, 1), full(1, 1),
        ],
        out_specs=pl.BlockSpec((R, 1), lambda i: (i, 0)),
        out_shape=jax.ShapeDtypeStruct((N, 1), jnp.float32),
    )


def kernel(x, edge_index, W_gcn, b_gcn, W1, b1, W2, b2, W3, b3, Wo, bo):
    N, D_IN = x.shape
    D = W_gcn.shape[1]
    DH = W1.shape[1]
    E = edge_index.shape[1]

    EP = _EPT * _NS
    pad = EP - E
    row_p = jnp.concatenate([edge_index[0], jnp.zeros((pad,), jnp.int32)])
    col_p = jnp.concatenate([edge_index[1], jnp.full((pad,), N, jnp.int32)])
    col = edge_index[1]
    ones_e = jnp.ones((E // (_NC * _NS),), jnp.float32)
    zeros_n = jnp.zeros((N,), jnp.float32)

    degp = _make_deg_kernel(N, E)(col, ones_e, zeros_n)
    hp_all, dis = _make_linear_kernel(N, D_IN, D)(x, W_gcn, degp.T)
    acc_all = _make_scatter_kernel(N, E, D)(row_p, col_p, hp_all)
    out = _make_mlp_kernel(N, D, DH)(
        acc_all, acc_all, acc_all, acc_all,
        hp_all, hp_all, hp_all, hp_all,
        dis, b_gcn.reshape(1, D),
        W1, b1.reshape(1, DH), W2, b2.reshape(1, DH),
        W3, b3.reshape(1, DH), Wo, bo.reshape(1, 1))
    return out


# speedup vs baseline: 8.1148x; 1.0001x over previous
import functools

import jax
import jax.numpy as jnp
from jax import lax
from jax.experimental import pallas as pl
from jax.experimental.pallas import tpu as pltpu
from jax.experimental.pallas import tpu_sc as plsc

ALPHA = 0.1

_NC = 2
_NS = 16
_L = 16

_SL = 128
_EPT = 20480


def _make_deg_kernel(N, E):
    EPW = E // (_NC * _NS)
    mesh = plsc.VectorSubcoreMesh(core_axis_name="c", subcore_axis_name="s")

    @functools.partial(
        pl.kernel,
        out_type=jax.ShapeDtypeStruct((_NC, N), jnp.float32),
        mesh=mesh,
        compiler_params=pltpu.CompilerParams(needs_layout_passes=False),
        scratch_types=[
            pltpu.VMEM((EPW,), jnp.int32),
            pltpu.VMEM((EPW,), jnp.float32),
            pltpu.VMEM_SHARED((N,), jnp.float32),
        ],
    )
    def deg_k(col_hbm, ones_hbm, zeros_hbm, out_hbm, col_v, ones_v, deg_sh):
        c = lax.axis_index("c")
        s = lax.axis_index("s")
        base = (c * _NS + s) * EPW
        pltpu.sync_copy(col_hbm.at[pl.ds(base, EPW)], col_v)
        pltpu.sync_copy(ones_hbm, ones_v)

        @pl.when(s == 0)
        def _():
            pltpu.sync_copy(zeros_hbm, deg_sh)

        plsc.subcore_barrier()
        pltpu.sync_copy(ones_v, deg_sh.at[col_v], add=True)
        plsc.subcore_barrier()

        @pl.when(s == 0)
        def _():
            pltpu.sync_copy(deg_sh, out_hbm.at[c])

    return deg_k


def _make_scatter_kernel(N, E, D):
    NSL = D // _SL
    SW = NSL // _NC
    B = 128
    STG = 2048
    NSTG = _EPT // STG
    NB = STG // B
    AR = N + 8
    RS = 640
    LAST = N - (_NS - 1) * RS
    mesh = plsc.VectorSubcoreMesh(core_axis_name="c", subcore_axis_name="s")

    @functools.partial(
        pl.kernel,
        out_type=jax.ShapeDtypeStruct((NSL * N, _SL), jnp.float32),
        mesh=mesh,
        compiler_params=pltpu.CompilerParams(needs_layout_passes=False),
        scratch_types=[
            pltpu.VMEM((STG,), jnp.int32),
            pltpu.VMEM((STG,), jnp.int32),
            pltpu.VMEM((B, _SL), jnp.float32),
            pltpu.VMEM((B, _SL), jnp.float32),
            pltpu.VMEM_SHARED((AR, _SL), jnp.float32),
            pltpu.SemaphoreType.DMA,
            pltpu.SemaphoreType.DMA,
            pltpu.SemaphoreType.DMA,
            pltpu.SemaphoreType.DMA,
        ],
    )
    def scat_k(row_hbm, col_hbm, hp_hbm, out_hbm,
               er_v, ec_v, rows_a, rows_b, acc_sh,
               sem_a, sem_b, sem_sa, sem_sb):
        c = lax.axis_index("c")
        s = lax.axis_index("s")
        zero16f = jnp.zeros((_L,), jnp.float32)

        for w in range(SW):
            sl_id = w * _NC + c

            def zb(i, _):
                for k in range(_SL // _L):
                    rows_a[i, pl.ds(k * _L, _L)] = zero16f
                return 0
            lax.fori_loop(0, B, zb, 0)

            @pl.when(s < _NS - 1)
            def _():
                for t in range(RS // B):
                    pltpu.sync_copy(rows_a, acc_sh.at[pl.ds(s * RS + t * B, B)])

            @pl.when(s == _NS - 1)
            def _():
                base = (_NS - 1) * RS
                nlast = LAST + 8
                for t in range(nlast // B):
                    pltpu.sync_copy(rows_a, acc_sh.at[pl.ds(base + t * B, B)])
                if nlast % B:
                    pltpu.sync_copy(
                        rows_a.at[pl.ds(0, nlast % B)],
                        acc_sh.at[pl.ds(base + (nlast // B) * B, nlast % B)])

            plsc.subcore_barrier()

            off16 = jnp.zeros((_L,), jnp.int32) + sl_id * N

            def sb(st, _):
                blk = s * _EPT + st * STG
                pltpu.sync_copy(row_hbm.at[pl.ds(blk, STG)], er_v)
                pltpu.sync_copy(col_hbm.at[pl.ds(blk, STG)], ec_v)

                def ob(i, _):
                    er_v[pl.ds(i * _L, _L)] = er_v[pl.ds(i * _L, _L)] + off16
                    return 0
                lax.fori_loop(0, STG // _L, ob, 0)

                bufs = (rows_a, rows_b)
                gsems = (sem_a, sem_b)
                ssems = (sem_sa, sem_sb)
                dg = [None, None]
                ds = [None, None]
                dg[0] = pltpu.async_copy(
                    hp_hbm.at[er_v.at[pl.ds(0, B)]], rows_a, sem_a)
                for k in range(NB):
                    p, q = k % 2, (k + 1) % 2
                    dg[p].wait()
                    ds[p] = pltpu.async_copy(
                        bufs[p], acc_sh.at[ec_v.at[pl.ds(k * B, B)]],
                        ssems[p], add=True)
                    if k + 1 < NB:
                        if ds[q] is not None:
                            ds[q].wait()
                        dg[q] = pltpu.async_copy(
                            hp_hbm.at[er_v.at[pl.ds((k + 1) * B, B)]],
                            bufs[q], gsems[q])
                ds[(NB - 1) % 2].wait()
                if ds[NB % 2] is not None:
                    ds[NB % 2].wait()
                return 0

            lax.fori_loop(0, NSTG, sb, 0)
            plsc.subcore_barrier()

            obase = sl_id * N

            @pl.when(s < _NS - 1)
            def _():
                pltpu.sync_copy(acc_sh.at[pl.ds(s * RS, RS)],
                                out_hbm.at[pl.ds(obase + s * RS, RS)])

            @pl.when(s == _NS - 1)
            def _():
                pltpu.sync_copy(acc_sh.at[pl.ds((_NS - 1) * RS, LAST)],
                                out_hbm.at[pl.ds(obase + (_NS - 1) * RS, LAST)])

            plsc.subcore_barrier()

    return scat_k


def _make_linear_kernel(N, D_IN, D):
    R = 1000
    NSL = D // _SL
    NR = N // R
    grid = (N // R, NSL)

    def body(x_ref, w_ref, degp_ref, hp_ref, dis_ref):
        deg = degp_ref[...]
        d = deg[:, 0:1] + deg[:, 1:2] + 1.0
        dis = lax.rsqrt(d)
        hp_ref[...] = jnp.dot(x_ref[...], w_ref[...],
                              preferred_element_type=jnp.float32) * dis
        dis_ref[...] = dis

    return pl.pallas_call(
        body,
        grid=grid,
        in_specs=[
            pl.BlockSpec((R, D_IN), lambda i, j: (i, 0)),
            pl.BlockSpec((D_IN, _SL), lambda i, j: (0, j)),
            pl.BlockSpec((R, 2), lambda i, j: (i, 0)),
        ],
        out_specs=[
            pl.BlockSpec((R, _SL), lambda i, j: (j * NR + i, 0)),
            pl.BlockSpec((R, 1), lambda i, j: (i, 0)),
        ],
        out_shape=[
            jax.ShapeDtypeStruct((NSL * N, _SL), jnp.float32),
            jax.ShapeDtypeStruct((N, 1), jnp.float32),
        ],
    )


def _make_mlp_kernel(N, D, DH):
    R = 1000
    NSL = D // _SL
    NR = N // R
    grid = (N // R,)

    def _leaky(z):
        return jnp.where(z >= 0, z, ALPHA * z)

    def body(a0, a1, a2, a3, h0, h1, h2, h3, dis_ref, bg_ref,
             w1_ref, b1_ref, w2_ref, b2_ref, w3_ref, b3_ref,
             wo_ref, bo_ref, out_ref):
        dis = dis_ref[...]
        acc = jnp.concatenate([a0[...], a1[...], a2[...], a3[...]], axis=1)
        hp = jnp.concatenate([h0[...], h1[...], h2[...], h3[...]], axis=1)
        conv = dis * (acc + hp) + bg_ref[...]
        z1 = _leaky(jnp.dot(conv, w1_ref[...],
                            preferred_element_type=jnp.float32) + b1_ref[...])
        z2 = _leaky(jnp.dot(z1, w2_ref[...],
                            preferred_element_type=jnp.float32) + b2_ref[...])
        z3 = _leaky(jnp.dot(z2, w3_ref[...],
                            preferred_element_type=jnp.float32) + b3_ref[...])
        out_ref[...] = jnp.dot(z3, wo_ref[...],
                               preferred_element_type=jnp.float32) + bo_ref[...]

    full = lambda a, b: pl.BlockSpec((a, b), lambda i: (0, 0))

    def slice_spec(k):
        return pl.BlockSpec((R, _SL), lambda i, k=k: (k * NR + i, 0))

    return pl.pallas_call(
        body,
        grid=grid,
        in_specs=[slice_spec(k) for k in range(NSL)]
        + [slice_spec(k) for k in range(NSL)]
        + [
            pl.BlockSpec((R, 1), lambda i: (i, 0)),
            full(1, D),
            full(D, DH), full(1, DH),
            full(DH, DH), full(1, DH),
            full(DH, DH), full(1, DH),
            full(DH, 1), full(1, 1),
        ],
        out_specs=pl.BlockSpec((R, 1), lambda i: (i, 0)),
        out_shape=jax.ShapeDtypeStruct((N, 1), jnp.float32),
    )


def kernel(x, edge_index, W_gcn, b_gcn, W1, b1, W2, b2, W3, b3, Wo, bo):
    N, D_IN = x.shape
    D = W_gcn.shape[1]
    DH = W1.shape[1]
    E = edge_index.shape[1]

    EP = _EPT * _NS
    pad = EP - E
    row_p = jnp.concatenate([edge_index[0], jnp.zeros((pad,), jnp.int32)])
    col_p = jnp.concatenate([edge_index[1], jnp.full((pad,), N, jnp.int32)])
    col = edge_index[1]
    ones_e = jnp.ones((E // (_NC * _NS),), jnp.float32)
    zeros_n = jnp.zeros((N,), jnp.float32)

    degp = _make_deg_kernel(N, E)(col, ones_e, zeros_n)
    hp_all, dis = _make_linear_kernel(N, D_IN, D)(x, W_gcn, degp.T)
    acc_all = _make_scatter_kernel(N, E, D)(row_p, col_p, hp_all)
    out = _make_mlp_kernel(N, D, DH)(
        acc_all, acc_all, acc_all, acc_all,
        hp_all, hp_all, hp_all, hp_all,
        dis, b_gcn.reshape(1, D),
        W1, b1.reshape(1, DH), W2, b2.reshape(1, DH),
        W3, b3.reshape(1, DH), Wo, bo.reshape(1, 1))
    return out
